# Initial kernel scaffold; baseline (speedup 1.0000x reference)
import jax
import jax.numpy as jnp
from jax.experimental import pallas as pl


def kernel(M, edge_index, rev_index, dim_size, a):
    raise NotImplementedError("write your pallas kernel here")


# speedup vs baseline: 2.5071x; 2.5071x over previous
import functools

import jax
import jax.numpy as jnp
from jax import lax
from jax.experimental import pallas as pl
from jax.experimental.pallas import tpu as pltpu
from jax.experimental.pallas import tpu_sc as plsc

NC = 2
NS = 16
L = 16
CH = 80
NEG = -3.0e38


def _score_tc(M, a, E, D):
    BE = 4096

    def body(m_ref, a_ref, o_ref):
        o_ref[...] = jnp.sum(m_ref[...] * a_ref[...][None, :], axis=1)

    return pl.pallas_call(
        body,
        grid=(pl.cdiv(E, BE),),
        in_specs=[
            pl.BlockSpec((BE, D), lambda i: (i, 0)),
            pl.BlockSpec((D,), lambda i: (0,)),
        ],
        out_specs=pl.BlockSpec((BE,), lambda i: (i,)),
        out_shape=jax.ShapeDtypeStruct((E,), jnp.float32),
    )(M, a)


def _weighted_tc(M, alpha, E, D):
    BE = 4096
    H = D // 2

    def body(m_ref, al_ref, lo_ref, hi_ref):
        w = m_ref[...] * al_ref[...][:, None]
        lo_ref[...] = w[:, :H]
        hi_ref[...] = w[:, H:]

    return pl.pallas_call(
        body,
        grid=(pl.cdiv(E, BE),),
        in_specs=[
            pl.BlockSpec((BE, D), lambda i: (i, 0)),
            pl.BlockSpec((BE,), lambda i: (i,)),
        ],
        out_specs=[
            pl.BlockSpec((BE, H), lambda i: (i, 0)),
            pl.BlockSpec((BE, H), lambda i: (i, 0)),
        ],
        out_shape=[
            jax.ShapeDtypeStruct((E, H), jnp.float32),
            jax.ShapeDtypeStruct((E, H), jnp.float32),
        ],
    )(M, alpha)


def _stats_sc(score, dest, E, Np):
    EPC = E // NS
    ECH = EPC // CH
    EW = E // (NC * NS)
    AW = EW // CH
    SEG = Np // NS

    mesh = plsc.VectorSubcoreMesh(core_axis_name="c", subcore_axis_name="s", num_cores=NC, num_subcores=NS)

    @functools.partial(
        pl.kernel,
        out_type=jax.ShapeDtypeStruct((E,), jnp.float32),
        mesh=mesh,
        compiler_params=pltpu.CompilerParams(needs_layout_passes=False),
        scratch_types=[
            pltpu.VMEM((CH,), jnp.float32),
            pltpu.VMEM((CH,), jnp.int32),
            pltpu.VMEM((CH,), jnp.float32),
            pltpu.VMEM((Np,), jnp.float32),
            pltpu.VMEM((Np,), jnp.float32),
            pltpu.VMEM((Np,), jnp.float32),
            pltpu.VMEM((SEG,), jnp.float32),
            pltpu.VMEM((SEG,), jnp.float32),
            pltpu.VMEM_SHARED((NS, Np), jnp.float32),
        ],
    )
    def k(score_hbm, dest_hbm, alpha_hbm,
          sc_v, d_v, a_v, priv, smax_full, den_full, comb_v, tmp_v, stage):
        cid = lax.axis_index("c")
        sid = lax.axis_index("s")
        n0 = sid * SEG

        @pl.loop(0, Np // L)
        def _(i):
            priv[pl.ds(i * L, L)] = jnp.full((L,), NEG, jnp.float32)

        @pl.loop(0, ECH)
        def _(ci):
            e0 = sid * EPC + ci * CH
            sl = pl.ds(e0, CH)
            pltpu.sync_copy(score_hbm.at[sl], sc_v)
            pltpu.sync_copy(dest_hbm.at[sl], d_v)

            @pl.loop(0, CH // L)
            def _(j):
                d = d_v[pl.ds(j * L, L)]
                s = sc_v[pl.ds(j * L, L)]
                cur = plsc.load_gather(priv, [d])

                def cond(c):
                    return jnp.any(c)

                def body(c):
                    plsc.store_scatter(priv, [d], s, mask=c)
                    return s > plsc.load_gather(priv, [d])

                lax.while_loop(cond, body, s > cur)

        pltpu.sync_copy(priv, stage.at[sid])
        plsc.subcore_barrier()
        pltpu.sync_copy(stage.at[0, pl.ds(n0, SEG)], comb_v)
        for r in range(1, NS):
            pltpu.sync_copy(stage.at[r, pl.ds(n0, SEG)], tmp_v)

            @pl.loop(0, SEG // L)
            def _(i):
                sl = pl.ds(i * L, L)
                comb_v[sl] = jnp.maximum(comb_v[sl], tmp_v[sl])

        plsc.subcore_barrier()
        pltpu.sync_copy(comb_v, stage.at[0, pl.ds(n0, SEG)])
        plsc.subcore_barrier()
        pltpu.sync_copy(stage.at[0], smax_full)
        plsc.subcore_barrier()

        @pl.loop(0, Np // L)
        def _(i):
            priv[pl.ds(i * L, L)] = jnp.zeros((L,), jnp.float32)

        @pl.loop(0, ECH)
        def _(ci):
            e0 = sid * EPC + ci * CH
            sl = pl.ds(e0, CH)
            pltpu.sync_copy(score_hbm.at[sl], sc_v)
            pltpu.sync_copy(dest_hbm.at[sl], d_v)

            @pl.loop(0, CH // L)
            def _(j):
                d = d_v[pl.ds(j * L, L)]
                s = sc_v[pl.ds(j * L, L)]
                sm = plsc.load_gather(smax_full, [d])
                plsc.addupdate_scatter(priv, [d], jnp.exp(s - sm))

        pltpu.sync_copy(priv, stage.at[sid])
        plsc.subcore_barrier()
        pltpu.sync_copy(stage.at[0, pl.ds(n0, SEG)], comb_v)
        for r in range(1, NS):
            pltpu.sync_copy(stage.at[r, pl.ds(n0, SEG)], tmp_v)

            @pl.loop(0, SEG // L)
            def _(i):
                sl = pl.ds(i * L, L)
                comb_v[sl] = comb_v[sl] + tmp_v[sl]

        plsc.subcore_barrier()
        pltpu.sync_copy(comb_v, stage.at[0, pl.ds(n0, SEG)])
        plsc.subcore_barrier()
        pltpu.sync_copy(stage.at[0], den_full)

        @pl.loop(0, AW)
        def _(ci):
            e0 = (cid * NS + sid) * EW + ci * CH
            sl = pl.ds(e0, CH)
            pltpu.sync_copy(score_hbm.at[sl], sc_v)
            pltpu.sync_copy(dest_hbm.at[sl], d_v)

            @pl.loop(0, CH // L)
            def _(j):
                d = d_v[pl.ds(j * L, L)]
                s = sc_v[pl.ds(j * L, L)]
                sm = plsc.load_gather(smax_full, [d])
                dn = plsc.load_gather(den_full, [d])
                a_v[pl.ds(j * L, L)] = jnp.exp(s - sm) / (dn + 1e-16)

            pltpu.sync_copy(a_v, alpha_hbm.at[sl])

    return k(score, dest)


def _agg_sc(wlo, whi, dest, src, rev, E, Np, D):
    H = D // 2
    EPC = E // NS
    ECH = EPC // CH
    ZR = Np // NS // CH

    mesh = plsc.VectorSubcoreMesh(core_axis_name="c", subcore_axis_name="s", num_cores=NC, num_subcores=NS)

    @functools.partial(
        pl.kernel,
        out_type=jax.ShapeDtypeStruct((E, D), jnp.float32),
        mesh=mesh,
        compiler_params=pltpu.CompilerParams(
            needs_layout_passes=False, use_tc_tiling_on_sc=False),
        scratch_types=[
            pltpu.VMEM((CH, H), jnp.float32),
            pltpu.VMEM((CH,), jnp.int32),
            pltpu.VMEM((CH,), jnp.int32),
            pltpu.VMEM((CH,), jnp.int32),
            pltpu.VMEM((CH, H), jnp.float32),
            pltpu.VMEM((CH, H), jnp.float32),
            pltpu.VMEM((CH, H), jnp.float32),
            pltpu.VMEM_SHARED((Np, H), jnp.float32),
        ],
    )
    def k(wlo_hbm, whi_hbm, dest_hbm, src_hbm, rev_hbm, out_hbm,
          rows_v, d_v, src_v, rev_v, mv_v, wr_v, o_v, mv_s):
        cid = lax.axis_index("c")
        sid = lax.axis_index("s")

        @pl.loop(0, CH)
        def _(r):
            for g in range(H // L):
                o_v[r, pl.ds(g * L, L)] = jnp.zeros((L,), jnp.float32)

        @pl.loop(0, ZR)
        def _(i):
            pltpu.sync_copy(o_v, mv_s.at[pl.ds((sid * ZR + i) * CH, CH)])

        plsc.subcore_barrier()

        @pl.loop(0, ECH)
        def _(ci):
            e0 = sid * EPC + ci * CH
            sl = pl.ds(e0, CH)

            @pl.when(cid == 0)
            def _():
                pltpu.sync_copy(wlo_hbm.at[sl], rows_v)

            @pl.when(cid == 1)
            def _():
                pltpu.sync_copy(whi_hbm.at[sl], rows_v)

            pltpu.sync_copy(dest_hbm.at[sl], d_v)
            pltpu.sync_copy(rows_v, mv_s.at[d_v], add=True)

        plsc.subcore_barrier()

        @pl.loop(0, ECH)
        def _(ci):
            e0 = sid * EPC + ci * CH
            sl = pl.ds(e0, CH)
            pltpu.sync_copy(src_hbm.at[sl], src_v)
            pltpu.sync_copy(rev_hbm.at[sl], rev_v)
            pltpu.sync_copy(mv_s.at[src_v], mv_v)

            @pl.when(cid == 0)
            def _():
                pltpu.sync_copy(wlo_hbm.at[rev_v], wr_v)

            @pl.when(cid == 1)
            def _():
                pltpu.sync_copy(whi_hbm.at[rev_v], wr_v)

            @pl.loop(0, CH)
            def _(r):
                for g in range(H // L):
                    csl = pl.ds(g * L, L)
                    o_v[r, csl] = mv_v[r, csl] - wr_v[r, csl]

            @pl.when(cid == 0)
            def _():
                pltpu.sync_copy(o_v, out_hbm.at[sl, pl.ds(0, H)])

            @pl.when(cid == 1)
            def _():
                pltpu.sync_copy(o_v, out_hbm.at[sl, pl.ds(H, H)])

    return k(wlo, whi, dest, src, rev)


def kernel(M, edge_index, rev_index, dim_size, a):
    E, D = M.shape
    Np = 10240
    src = edge_index[0]
    dest = edge_index[1]
    score = _score_tc(M, a, E, D)
    alpha = _stats_sc(score, dest, E, Np)
    wlo, whi = _weighted_tc(M, alpha, E, D)
    return _agg_sc(wlo, whi, dest, src, rev_index, E, Np, D)


# speedup vs baseline: 3.6116x; 1.4405x over previous
import functools

import jax
import jax.numpy as jnp
from jax import lax
from jax.experimental import pallas as pl
from jax.experimental.pallas import tpu as pltpu
from jax.experimental.pallas import tpu_sc as plsc

NC = 2
NS = 16
L = 16
CH = 80
SEC = 4000
NEG = -3.0e38


def _score_tc(M, a, E, D):
    BE = 4096

    def body(m_ref, a_ref, o_ref):
        o_ref[...] = jnp.sum(m_ref[...] * a_ref[...][None, :], axis=1)

    return pl.pallas_call(
        body,
        grid=(pl.cdiv(E, BE),),
        in_specs=[
            pl.BlockSpec((BE, D), lambda i: (i, 0)),
            pl.BlockSpec((D,), lambda i: (0,)),
        ],
        out_specs=pl.BlockSpec((BE,), lambda i: (i,)),
        out_shape=jax.ShapeDtypeStruct((E,), jnp.float32),
    )(M, a)


def _mega_sc(M, score, dest, src, rev, E, Np, D):
    H = D // 2
    EPC = E // NS
    ECH = EPC // CH
    NSEC = EPC // SEC
    SEG = Np // NS
    ZR = SEG // CH

    mesh = plsc.VectorSubcoreMesh(
        core_axis_name="c", subcore_axis_name="s",
        num_cores=NC, num_subcores=NS)

    @functools.partial(
        pl.kernel,
        out_type=(jax.ShapeDtypeStruct((E, D), jnp.float32),
                  jax.ShapeDtypeStruct((E, H), jnp.float32),
                  jax.ShapeDtypeStruct((E, H), jnp.float32)),
        mesh=mesh,
        compiler_params=pltpu.CompilerParams(
            needs_layout_passes=False, use_tc_tiling_on_sc=False),
        scratch_types=[
            pltpu.VMEM((SEC,), jnp.float32),
            pltpu.VMEM((SEC,), jnp.int32),
            pltpu.VMEM((Np,), jnp.float32),
            pltpu.VMEM((Np,), jnp.float32),
            pltpu.VMEM((SEG,), jnp.float32),
            pltpu.VMEM((SEG,), jnp.float32),
            pltpu.VMEM((CH,), jnp.float32),
            pltpu.VMEM((CH,), jnp.float32),
            pltpu.VMEM((CH,), jnp.int32),
            pltpu.VMEM((CH, H), jnp.float32),
            pltpu.VMEM((CH, H), jnp.float32),
            pltpu.VMEM((CH,), jnp.int32),
            pltpu.VMEM((CH,), jnp.int32),
            pltpu.VMEM((CH, H), jnp.float32),
            pltpu.VMEM((CH, H), jnp.float32),
            pltpu.VMEM((CH, H), jnp.float32),
            pltpu.VMEM_SHARED((NS, Np), jnp.float32),
            pltpu.VMEM_SHARED((Np, H), jnp.float32),
            pltpu.SemaphoreType.DMA,
            pltpu.SemaphoreType.DMA,
        ],
    )
    def k(m_hbm, score_hbm, dest_hbm, src_hbm, rev_hbm,
          out_hbm, wlo_hbm, whi_hbm,
          sc_sec, d_sec, smax, den, comb_v, tmp_v,
          sc_v, al_v, d_v, rows0, rows1, src_v, rev_v, mv_v, wr_v, o_v,
          stage, mv_s, sem0, sem1):
        cid = lax.axis_index("c")
        sid = lax.axis_index("s")
        n0 = sid * SEG
        e_base = sid * EPC

        @pl.loop(0, CH)
        def _(r):
            for g in range(H // L):
                o_v[r, pl.ds(g * L, L)] = jnp.zeros((L,), jnp.float32)

        @pl.loop(0, ZR)
        def _(i):
            pltpu.sync_copy(o_v, mv_s.at[pl.ds(n0 + i * CH, CH)])

        @pl.loop(0, Np // L)
        def _(i):
            den[pl.ds(i * L, L)] = jnp.full((L,), NEG, jnp.float32)

        @pl.loop(0, NSEC)
        def _(sec):
            ssl = pl.ds(e_base + sec * SEC, SEC)
            pltpu.sync_copy(score_hbm.at[ssl], sc_sec)
            pltpu.sync_copy(dest_hbm.at[ssl], d_sec)

            @pl.loop(0, SEC // L)
            def _(j):
                d = d_sec[pl.ds(j * L, L)]
                s = sc_sec[pl.ds(j * L, L)]
                cur = plsc.load_gather(den, [d])

                def cond(c):
                    return jnp.any(c)

                def body(c):
                    plsc.store_scatter(den, [d], s, mask=c)
                    return s > plsc.load_gather(den, [d])

                lax.while_loop(cond, body, s > cur)

        pltpu.sync_copy(den, stage.at[sid])
        plsc.subcore_barrier()
        pltpu.sync_copy(stage.at[0, pl.ds(n0, SEG)], comb_v)
        for r in range(1, NS):
            pltpu.sync_copy(stage.at[r, pl.ds(n0, SEG)], tmp_v)

            @pl.loop(0, SEG // L)
            def _(i):
                sl = pl.ds(i * L, L)
                comb_v[sl] = jnp.maximum(comb_v[sl], tmp_v[sl])

        plsc.subcore_barrier()
        pltpu.sync_copy(comb_v, stage.at[0, pl.ds(n0, SEG)])
        plsc.subcore_barrier()
        pltpu.sync_copy(stage.at[0], smax)
        plsc.subcore_barrier()

        @pl.loop(0, Np // L)
        def _(i):
            den[pl.ds(i * L, L)] = jnp.zeros((L,), jnp.float32)

        @pl.loop(0, NSEC)
        def _(sec):
            ssl = pl.ds(e_base + sec * SEC, SEC)
            pltpu.sync_copy(score_hbm.at[ssl], sc_sec)
            pltpu.sync_copy(dest_hbm.at[ssl], d_sec)

            @pl.loop(0, SEC // L)
            def _(j):
                d = d_sec[pl.ds(j * L, L)]
                s = sc_sec[pl.ds(j * L, L)]
                sm = plsc.load_gather(smax, [d])
                plsc.addupdate_scatter(den, [d], jnp.exp(s - sm))

        pltpu.sync_copy(den, stage.at[sid])
        plsc.subcore_barrier()
        pltpu.sync_copy(stage.at[0, pl.ds(n0, SEG)], comb_v)
        for r in range(1, NS):
            pltpu.sync_copy(stage.at[r, pl.ds(n0, SEG)], tmp_v)

            @pl.loop(0, SEG // L)
            def _(i):
                sl = pl.ds(i * L, L)
                comb_v[sl] = comb_v[sl] + tmp_v[sl]

        plsc.subcore_barrier()
        pltpu.sync_copy(comb_v, stage.at[0, pl.ds(n0, SEG)])
        plsc.subcore_barrier()
        pltpu.sync_copy(stage.at[0], den)

        def start_load(c, buf, sem):
            sl = pl.ds(e_base + c * CH, CH)

            @pl.when(cid == 0)
            def _():
                pltpu.async_copy(m_hbm.at[sl, pl.ds(0, H)], buf, sem)

            @pl.when(cid == 1)
            def _():
                pltpu.async_copy(m_hbm.at[sl, pl.ds(H, H)], buf, sem)

        def wait_load(buf, sem):
            pltpu.make_async_copy(
                m_hbm.at[pl.ds(0, CH), pl.ds(0, H)], buf, sem).wait()

        def process(c, buf):
            sl = pl.ds(e_base + c * CH, CH)
            pltpu.sync_copy(score_hbm.at[sl], sc_v)
            pltpu.sync_copy(dest_hbm.at[sl], d_v)

            @pl.loop(0, CH // L)
            def _(jj):
                d = d_v[pl.ds(jj * L, L)]
                s = sc_v[pl.ds(jj * L, L)]
                sm = plsc.load_gather(smax, [d])
                dn = plsc.load_gather(den, [d])
                al_v[pl.ds(jj * L, L)] = jnp.exp(s - sm) / (dn + 1e-16)

            @pl.loop(0, CH // L)
            def _(jj):
                alv = al_v[pl.ds(jj * L, L)]
                for r16 in range(L):
                    a_s = alv[r16]
                    row = jj * L + r16
                    for g in range(H // L):
                        rsl = pl.ds(g * L, L)
                        buf[row, rsl] = buf[row, rsl] * a_s

            @pl.when(cid == 0)
            def _():
                pltpu.sync_copy(buf, wlo_hbm.at[sl])

            @pl.when(cid == 1)
            def _():
                pltpu.sync_copy(buf, whi_hbm.at[sl])

            pltpu.sync_copy(buf, mv_s.at[d_v], add=True)

        start_load(0, rows0, sem0)

        @pl.loop(0, ECH // 2)
        def _(go):
            c0 = go * 2
            wait_load(rows0, sem0)
            start_load(c0 + 1, rows1, sem1)
            process(c0, rows0)
            wait_load(rows1, sem1)

            @pl.when(c0 + 2 < ECH)
            def _():
                start_load(c0 + 2, rows0, sem0)

            process(c0 + 1, rows1)

        plsc.subcore_barrier()

        @pl.loop(0, ECH)
        def _(c):
            sl = pl.ds(e_base + c * CH, CH)
            pltpu.sync_copy(src_hbm.at[sl], src_v)
            pltpu.sync_copy(rev_hbm.at[sl], rev_v)

            @pl.when(cid == 0)
            def _():
                pltpu.async_copy(wlo_hbm.at[rev_v], wr_v, sem0)

            @pl.when(cid == 1)
            def _():
                pltpu.async_copy(whi_hbm.at[rev_v], wr_v, sem0)

            pltpu.sync_copy(mv_s.at[src_v], mv_v)
            pltpu.make_async_copy(
                wlo_hbm.at[pl.ds(0, CH)], wr_v, sem0).wait()

            @pl.loop(0, CH)
            def _(r):
                for g in range(H // L):
                    csl = pl.ds(g * L, L)
                    o_v[r, csl] = mv_v[r, csl] - wr_v[r, csl]

            @pl.when(cid == 0)
            def _():
                pltpu.sync_copy(o_v, out_hbm.at[sl, pl.ds(0, H)])

            @pl.when(cid == 1)
            def _():
                pltpu.sync_copy(o_v, out_hbm.at[sl, pl.ds(H, H)])

    return k(M, score, dest, src, rev)


def kernel(M, edge_index, rev_index, dim_size, a):
    E, D = M.shape
    Np = 10240
    src = edge_index[0]
    dest = edge_index[1]
    score = _score_tc(M, a, E, D)
    out, _, _ = _mega_sc(M, score, dest, src, rev_index, E, Np, D)
    return out
